# baseline (speedup 1.0000x reference)
import jax
import jax.numpy as jnp
from jax.experimental import pallas as pl
from jax.experimental.pallas import tpu as pltpu
from jax.experimental.pallas import tpu_sc as plsc

M_TILE = 2048
GATHER_WINDOW = 128


def _argmin_body(x_ref, w_ref, b_ref, o_ref):
    xb = x_ref[...].astype(jnp.bfloat16)
    wb = w_ref[...].astype(jnp.bfloat16)
    dot = jax.lax.dot_general(
        xb, wb, (((1,), (1,)), ((), ())),
        preferred_element_type=jnp.float32)
    dists = -2.0 * dot + b_ref[0, :][None, :]
    o_ref[0, 0, :] = jnp.argmin(dists, axis=1).astype(jnp.int32)


def _tc_argmin(x, weight, bias):
    m, _ = x.shape
    num_blocks = m // M_TILE
    out = pl.pallas_call(
        _argmin_body,
        grid=(num_blocks,),
        in_specs=[
            pl.BlockSpec((M_TILE, x.shape[1]), lambda i: (i, 0)),
            pl.BlockSpec(weight.shape, lambda i: (0, 0)),
            pl.BlockSpec(bias.shape, lambda i: (0, 0)),
        ],
        out_specs=pl.BlockSpec((1, 1, M_TILE), lambda i: (i, 0, 0)),
        out_shape=jax.ShapeDtypeStruct((num_blocks, 1, M_TILE), jnp.int32),
    )(x, weight, bias)
    return out.reshape(m)


def _sc_gather(weight, idxes):
    n = idxes.shape[0]
    indices = idxes.reshape(1, n)
    mesh = plsc.VectorSubcoreMesh(core_axis_name="core",
                                  subcore_axis_name="subcore")

    @pl.kernel(out_type=jax.ShapeDtypeStruct((n, weight.shape[1]),
                                             weight.dtype),
               mesh=mesh)
    def kern(w_hbm, i_hbm, o_hbm):
        def body(i_vmem, o_vmem):
            pltpu.sync_copy(w_hbm.at[i_vmem.at[0]], o_vmem)

        pltpu.emit_pipeline(
            body,
            grid=(n // GATHER_WINDOW,),
            in_specs=[pl.BlockSpec((1, GATHER_WINDOW),
                                   index_map=lambda i: (0, i))],
            out_specs=[pl.BlockSpec((GATHER_WINDOW, weight.shape[1]),
                                    index_map=lambda i: (i, 0))],
            core_axis_name=("core", "subcore"),
            dimension_semantics=(pltpu.PARALLEL,),
        )(i_hbm, o_hbm)

    return kern(weight, indices)


def kernel(input, weight):
    embed_dim = input.shape[-1]
    x = input.reshape(-1, embed_dim)
    bias = (weight * weight).sum(0)[None, :]
    idxes = _tc_argmin(x, weight, bias)
    values = _sc_gather(weight, idxes)
    return values.reshape(input.shape)
